# baseline (speedup 1.0000x reference)
import functools
import jax
import jax.numpy as jnp
from jax.experimental import pallas as pl

NUM_CLASSES = 8
MAX_TOTAL_SIZE = 100
NMS_IOU_THRESHOLD = 0.5
SCORE_THRESHOLD = 0.05
PRE_NMS_NUM_BOXES = 1000
BBOX_XFORM_CLIP = 4.135166556742356


def _decode_kernel(box_ref, cls_ref, anc_ref, img_ref, boxes_out, scores_out):
    deltas = box_ref[0]
    anchors = anc_ref[0]
    ya = (anchors[0] + anchors[2]) * 0.5
    xa = (anchors[1] + anchors[3]) * 0.5
    ha = anchors[2] - anchors[0]
    wa = anchors[3] - anchors[1]
    ty, tx = deltas[0], deltas[1]
    th = jnp.clip(deltas[2], -BBOX_XFORM_CLIP, BBOX_XFORM_CLIP)
    tw = jnp.clip(deltas[3], -BBOX_XFORM_CLIP, BBOX_XFORM_CLIP)
    h = jnp.exp(th) * ha
    w = jnp.exp(tw) * wa
    yc = ty * ha + ya
    xc = tx * wa + xa
    y1 = yc - h * 0.5
    x1 = xc - w * 0.5
    y2 = yc + h * 0.5
    x2 = xc + w * 0.5
    bi = pl.program_id(0)
    ih = img_ref[bi, 0]
    iw = img_ref[bi, 1]
    y1 = jnp.minimum(jnp.maximum(y1, 0.0), ih)
    x1 = jnp.minimum(jnp.maximum(x1, 0.0), iw)
    y2 = jnp.minimum(jnp.maximum(y2, 0.0), ih)
    x2 = jnp.minimum(jnp.maximum(x2, 0.0), iw)
    boxes_out[0] = jnp.stack([y1, x1, y2, x2], axis=0)
    scores_out[0] = jax.nn.sigmoid(cls_ref[0])


def _decode_pallas(box_t, cls_t, anc_t, image_shape):
    b, _, n = box_t.shape
    c = cls_t.shape[1]
    return pl.pallas_call(
        _decode_kernel,
        out_shape=(
            jax.ShapeDtypeStruct((b, 4, n), jnp.float32),
            jax.ShapeDtypeStruct((b, c, n), jnp.float32),
        ),
        grid=(b,),
        in_specs=[
            pl.BlockSpec((1, 4, n), lambda i: (i, 0, 0)),
            pl.BlockSpec((1, c, n), lambda i: (i, 0, 0)),
            pl.BlockSpec((1, 4, n), lambda i: (i, 0, 0)),
            pl.BlockSpec((b, 2), lambda i: (0, 0)),
        ],
        out_specs=(
            pl.BlockSpec((1, 4, n), lambda i: (i, 0, 0)),
            pl.BlockSpec((1, c, n), lambda i: (i, 0, 0)),
        ),
    )(box_t, cls_t, anc_t, image_shape)


def _iou_one_vs_all(box, boxes):
    yy1 = jnp.maximum(box[0], boxes[:, 0])
    xx1 = jnp.maximum(box[1], boxes[:, 1])
    yy2 = jnp.minimum(box[2], boxes[:, 2])
    xx2 = jnp.minimum(box[3], boxes[:, 3])
    inter = jnp.maximum(yy2 - yy1, 0.0) * jnp.maximum(xx2 - xx1, 0.0)
    a1 = (box[2] - box[0]) * (box[3] - box[1])
    a2 = (boxes[:, 2] - boxes[:, 0]) * (boxes[:, 3] - boxes[:, 1])
    return inter / (a1 + a2 - inter + 1e-8)


def _sorted_nms_padded(scores, boxes):
    suppressed0 = scores < 0.0
    def step(sup, _):
        avail = jnp.where(sup, -jnp.inf, scores)
        idx = jnp.argmax(avail)
        valid = avail[idx] > -jnp.inf
        box = jnp.where(valid, boxes[idx], jnp.zeros((4,), boxes.dtype))
        score = jnp.where(valid, scores[idx], -1.0)
        ious = _iou_one_vs_all(box, boxes)
        sup = sup | (ious > NMS_IOU_THRESHOLD)
        sup = sup.at[idx].set(True)
        return sup, (box, score)
    _, (sel_boxes, sel_scores) = jax.lax.scan(step, suppressed0, None, length=MAX_TOTAL_SIZE)
    return sel_scores, sel_boxes


@jax.jit
def kernel(box_outputs, class_outputs, anchor_boxes, image_shape):
    b = box_outputs.shape[0]
    nc = NUM_CLASSES
    boxes_t, scores_t = _decode_pallas(
        box_outputs.transpose(0, 2, 1), class_outputs.transpose(0, 2, 1),
        anchor_boxes.transpose(0, 2, 1), image_shape)
    boxes = boxes_t.transpose(0, 2, 1)
    scores_all = scores_t.transpose(0, 2, 1)[..., 1:]
    k = min(boxes.shape[1], PRE_NMS_NUM_BOXES)
    nmsed_boxes, nmsed_scores, nmsed_classes = [], [], []
    for i in range(nc - 1):
        s_i, idx = jax.lax.top_k(scores_all[:, :, i], k)
        b_i = jnp.take_along_axis(boxes, idx[..., None], axis=1)
        mask = s_i >= SCORE_THRESHOLD
        s_i = jnp.where(mask, s_i, -1.0)
        b_i = jnp.where(mask[..., None], b_i, 0.0)
        ns, nb = jax.vmap(_sorted_nms_padded)(s_i, b_i)
        nmsed_boxes.append(nb)
        nmsed_scores.append(ns)
        nmsed_classes.append(jnp.full((b, MAX_TOTAL_SIZE), i, dtype=jnp.int32))
    boxes_c = jnp.concatenate(nmsed_boxes, axis=1)
    scores_c = jnp.concatenate(nmsed_scores, axis=1)
    classes_c = jnp.concatenate(nmsed_classes, axis=1)
    final_scores, idx = jax.lax.top_k(scores_c, MAX_TOTAL_SIZE)
    final_boxes = jnp.take_along_axis(boxes_c, idx[..., None], axis=1)
    final_classes = jnp.take_along_axis(classes_c, idx, axis=1)
    valid_detections = jnp.sum((final_scores > -1.0).astype(jnp.int32), axis=1)
    return final_boxes, final_scores, final_classes, valid_detections


# speedup vs baseline: 3.9098x; 3.9098x over previous
import functools
import jax
import jax.numpy as jnp
from jax import lax
from jax.experimental import pallas as pl

NUM_CLASSES = 8
MAX_TOTAL_SIZE = 100
NMS_IOU_THRESHOLD = 0.5
SCORE_THRESHOLD = 0.05
PRE_NMS_NUM_BOXES = 1000
BBOX_XFORM_CLIP = 4.135166556742356

P = 56
T = 128
NB = 8
KC = NB * T
NMERGE = 704


def _decode_kernel(box_ref, cls_ref, anc_ref, img_ref, boxes_out, scores_out):
    deltas = box_ref[0]
    anchors = anc_ref[0]
    ya = (anchors[0] + anchors[2]) * 0.5
    xa = (anchors[1] + anchors[3]) * 0.5
    ha = anchors[2] - anchors[0]
    wa = anchors[3] - anchors[1]
    ty, tx = deltas[0], deltas[1]
    th = jnp.clip(deltas[2], -BBOX_XFORM_CLIP, BBOX_XFORM_CLIP)
    tw = jnp.clip(deltas[3], -BBOX_XFORM_CLIP, BBOX_XFORM_CLIP)
    h = jnp.exp(th) * ha
    w = jnp.exp(tw) * wa
    yc = ty * ha + ya
    xc = tx * wa + xa
    y1 = yc - h * 0.5
    x1 = xc - w * 0.5
    y2 = yc + h * 0.5
    x2 = xc + w * 0.5
    bi = pl.program_id(0)
    ih = img_ref[bi, 0]
    iw = img_ref[bi, 1]
    y1 = jnp.minimum(jnp.maximum(y1, 0.0), ih)
    x1 = jnp.minimum(jnp.maximum(x1, 0.0), iw)
    y2 = jnp.minimum(jnp.maximum(y2, 0.0), ih)
    x2 = jnp.minimum(jnp.maximum(x2, 0.0), iw)
    boxes_out[0] = jnp.stack([y1, x1, y2, x2], axis=0)
    scores_out[0] = jax.nn.sigmoid(cls_ref[0])


def _decode_pallas(box_t, cls_t, anc_t, image_shape):
    b, _, n = box_t.shape
    c = cls_t.shape[1]
    return pl.pallas_call(
        _decode_kernel,
        out_shape=(
            jax.ShapeDtypeStruct((b, 4, n), jnp.float32),
            jax.ShapeDtypeStruct((b, c, n), jnp.float32),
        ),
        grid=(b,),
        in_specs=[
            pl.BlockSpec((1, 4, n), lambda i: (i, 0, 0)),
            pl.BlockSpec((1, c, n), lambda i: (i, 0, 0)),
            pl.BlockSpec((1, 4, n), lambda i: (i, 0, 0)),
            pl.BlockSpec((b, 2), lambda i: (0, 0)),
        ],
        out_specs=(
            pl.BlockSpec((1, 4, n), lambda i: (i, 0, 0)),
            pl.BlockSpec((1, c, n), lambda i: (i, 0, 0)),
        ),
    )(box_t, cls_t, anc_t, image_shape)


def _iou(ay1, ax1, ay2, ax2, by1, bx1, by2, bx2):
    yy1 = jnp.maximum(ay1, by1)
    xx1 = jnp.maximum(ax1, bx1)
    yy2 = jnp.minimum(ay2, by2)
    xx2 = jnp.minimum(ax2, bx2)
    inter = jnp.maximum(yy2 - yy1, 0.0) * jnp.maximum(xx2 - xx1, 0.0)
    a1 = (ay2 - ay1) * (ax2 - ax1)
    a2 = (by2 - by1) * (bx2 - bx1)
    return inter / (a1 + a2 - inter + 1e-8)


def _nms_kernel(s_ref, y1_ref, x1_ref, y2_ref, x2_ref,
                y1c_ref, x1c_ref, y2c_ref, x2c_ref,
                ss_ref, sy1_ref, sx1_ref, sy2_ref, sx2_ref):
    iota_k = lax.broadcasted_iota(jnp.int32, (T, T, 1), 0)
    iota_i = lax.broadcasted_iota(jnp.int32, (T, T, 1), 1)
    tri_ki = iota_i < iota_k
    iota_o = lax.broadcasted_iota(jnp.int32, (T, 1, 1), 0).astype(jnp.float32)

    def blk_cond(st):
        j, cnt, _, _, _, _, _ = st
        return (j < NB) & jnp.any(cnt < float(MAX_TOTAL_SIZE))

    def blk_body(st):
        j, cnt, ss, sy1, sx1, sy2, sx2 = st
        sb = s_ref[j]
        by1, bx1, by2, bx2 = y1_ref[j], x1_ref[j], y2_ref[j], x2_ref[j]
        cy1, cx1, cy2, cx2 = y1c_ref[j], x1c_ref[j], y2c_ref[j], x2c_ref[j]
        active0 = sb >= 0.0
        iou_ps = _iou(cy1, cx1, cy2, cx2,
                      sy1[None], sx1[None], sy2[None], sx2[None])
        active0 = active0 & ~jnp.any(iou_ps > NMS_IOU_THRESHOLD, axis=1)
        iou_bb = _iou(cy1, cx1, cy2, cx2,
                      by1[None], bx1[None], by2[None], bx2[None])
        can_supf = jnp.where((iou_bb > NMS_IOU_THRESHOLD) & tri_ki, 1.0, 0.0)

        def f_cond(fs):
            _, changed, it = fs
            return (changed > 0) & (it < T)

        def f_body(fs):
            Sf, _, it = fs
            supped = jnp.max(Sf[None] * can_supf, axis=1)
            Snf = jnp.where(active0 & (supped <= 0.0), 1.0, 0.0)
            chg = jnp.any(Snf != Sf).astype(jnp.int32)
            return Snf, chg, it + 1

        Sf, _, _ = lax.while_loop(
            f_cond, f_body,
            (jnp.where(active0, 1.0, 0.0), jnp.int32(1), jnp.int32(0)))
        S = Sf > 0.0
        pos_in = jnp.sum(jnp.where(tri_ki, Sf[None], 0.0), axis=1)
        pos = cnt + pos_in
        emit = S & (pos < float(MAX_TOTAL_SIZE))
        ohf = jnp.where(emit[None] & (pos[None] == iota_o), 1.0, 0.0)
        hit = jnp.sum(ohf, axis=1) > 0.0

        def emit_val(v):
            return jnp.sum(ohf * v[None], axis=1)

        ss = jnp.where(hit, emit_val(sb), ss)
        sy1 = jnp.where(hit, emit_val(by1), sy1)
        sx1 = jnp.where(hit, emit_val(bx1), sx1)
        sy2 = jnp.where(hit, emit_val(by2), sy2)
        sx2 = jnp.where(hit, emit_val(bx2), sx2)
        cnt = cnt + jnp.sum(Sf, axis=0, keepdims=True)
        return j + 1, cnt, ss, sy1, sx1, sy2, sx2

    init = (jnp.int32(0), jnp.zeros((1, P), jnp.float32),
            jnp.full((T, P), -1.0, jnp.float32),
            jnp.zeros((T, P), jnp.float32), jnp.zeros((T, P), jnp.float32),
            jnp.zeros((T, P), jnp.float32), jnp.zeros((T, P), jnp.float32))
    _, _, ss, sy1, sx1, sy2, sx2 = lax.while_loop(blk_cond, blk_body, init)
    ss_ref[...] = ss
    sy1_ref[...] = sy1
    sx1_ref[...] = sx1
    sy2_ref[...] = sy2
    sx2_ref[...] = sx2


def _nms_pallas(s3, planes3):
    planes4 = [q[:, :, None, :] for q in planes3]
    out = jax.ShapeDtypeStruct((T, P), jnp.float32)
    return pl.pallas_call(
        _nms_kernel,
        out_shape=(out,) * 5,
    )(s3, *planes3, *planes4)


def _merge_kernel(srow_ref, scol_ref, y1_ref, x1_ref, y2_ref, x2_ref,
                  cls_ref, vals_ref, valid_ref):
    srow = srow_ref[0]
    scol = scol_ref[0]
    gt = jnp.where(srow > scol, 1.0, 0.0)
    iota_j = lax.broadcasted_iota(jnp.int32, (NMERGE, NMERGE), 1)
    iota_i = lax.broadcasted_iota(jnp.int32, (NMERGE, NMERGE), 0)
    tie = jnp.where((srow == scol) & (iota_j < iota_i), 1.0, 0.0)
    rank = jnp.sum(gt + tie, axis=1, keepdims=True)
    iota_o = lax.broadcasted_iota(jnp.int32, (1, T), 1).astype(jnp.float32)
    oh = jnp.where(rank == iota_o, 1.0, 0.0)

    def emit(vcol_ref):
        return jnp.sum(oh * vcol_ref[0], axis=0, keepdims=True)

    out_s = emit(scol_ref)
    vals_ref[0, 0:1, :] = out_s
    vals_ref[0, 1:2, :] = emit(y1_ref)
    vals_ref[0, 2:3, :] = emit(x1_ref)
    vals_ref[0, 3:4, :] = emit(y2_ref)
    vals_ref[0, 4:5, :] = emit(x2_ref)
    vals_ref[0, 5:6, :] = emit(cls_ref)
    nvalid = jnp.sum(jnp.where((out_s > -1.0) & (iota_o < float(MAX_TOTAL_SIZE)),
                               1.0, 0.0), axis=1, keepdims=True)
    valid_ref[0] = jnp.broadcast_to(nvalid, (1, T)).astype(jnp.int32)


def _merge_pallas(srow, scol, y1c, x1c, y2c, x2c, clsc):
    b = srow.shape[0]
    row_spec = pl.BlockSpec((1, 1, NMERGE), lambda i: (i, 0, 0))
    col_spec = pl.BlockSpec((1, NMERGE, 1), lambda i: (i, 0, 0))
    return pl.pallas_call(
        _merge_kernel,
        out_shape=(
            jax.ShapeDtypeStruct((b, 6, T), jnp.float32),
            jax.ShapeDtypeStruct((b, 1, T), jnp.int32),
        ),
        grid=(b,),
        in_specs=[row_spec] + [col_spec] * 6,
        out_specs=(
            pl.BlockSpec((1, 6, T), lambda i: (i, 0, 0)),
            pl.BlockSpec((1, 1, T), lambda i: (i, 0, 0)),
        ),
    )(srow, scol, y1c, x1c, y2c, x2c, clsc)


@jax.jit
def kernel(box_outputs, class_outputs, anchor_boxes, image_shape):
    b, n, _ = box_outputs.shape
    boxes_t, scores_t = _decode_pallas(
        box_outputs.transpose(0, 2, 1), class_outputs.transpose(0, 2, 1),
        anchor_boxes.transpose(0, 2, 1), image_shape)
    s_flat = scores_t[:, 1:, :].transpose(1, 0, 2).reshape(P, n)
    k = min(n, PRE_NMS_NUM_BOXES)
    topv, topi = lax.top_k(s_flat, k)
    mask = topv >= SCORE_THRESHOLD
    sv = jnp.where(mask, topv, -1.0)
    planes = []
    for d in range(4):
        pd = jnp.broadcast_to(boxes_t[None, :, d, :], (7, b, n)).reshape(P, n)
        pv = jnp.take_along_axis(pd, topi, axis=1)
        planes.append(jnp.where(mask, pv, 0.0))
    pad = KC - k
    sv = jnp.concatenate([sv, jnp.full((P, pad), -1.0, jnp.float32)], axis=1)
    planes = [jnp.concatenate([q, jnp.zeros((P, pad), jnp.float32)], axis=1)
              for q in planes]

    def to3(a):
        return a.reshape(P, NB, T).transpose(1, 2, 0)

    ss, sy1, sx1, sy2, sx2 = _nms_pallas(to3(sv), [to3(q) for q in planes])
    ss, sy1, sx1, sy2, sx2 = (a.T for a in (ss, sy1, sx1, sy2, sx2))

    def to_m(a):
        return a.reshape(7, b, T).transpose(1, 0, 2)[:, :, :MAX_TOTAL_SIZE] \
                .reshape(b, 7 * MAX_TOTAL_SIZE)

    s_m = to_m(ss)
    cls_m = jnp.broadcast_to(
        jnp.arange(7, dtype=jnp.float32)[None, :, None],
        (b, 7, MAX_TOTAL_SIZE)).reshape(b, 7 * MAX_TOTAL_SIZE)
    padm = NMERGE - 7 * MAX_TOTAL_SIZE
    s_m = jnp.concatenate([s_m, jnp.full((b, padm), -2.0, jnp.float32)], axis=1)

    def padz(a):
        return jnp.concatenate([a, jnp.zeros((b, padm), jnp.float32)], axis=1)

    cols = [s_m[:, :, None]] + \
           [padz(to_m(q))[:, :, None] for q in (sy1, sx1, sy2, sx2)] + \
           [padz(cls_m)[:, :, None]]
    vals, valid = _merge_pallas(s_m[:, None, :], *cols)
    final_scores = vals[:, 0, :MAX_TOTAL_SIZE]
    final_boxes = vals[:, 1:5, :MAX_TOTAL_SIZE].transpose(0, 2, 1)
    final_classes = vals[:, 5, :MAX_TOTAL_SIZE].astype(jnp.int32)
    valid_detections = valid[:, 0, 0]
    return final_boxes, final_scores, final_classes, valid_detections


# speedup vs baseline: 4.0045x; 1.0242x over previous
import functools
import jax
import jax.numpy as jnp
from jax import lax
from jax.experimental import pallas as pl
from jax.experimental.pallas import tpu as pltpu
from jax.experimental.pallas import tpu_sc as plsc

NUM_CLASSES = 8
MAX_TOTAL_SIZE = 100
NMS_IOU_THRESHOLD = 0.5
SCORE_THRESHOLD = 0.05
PRE_NMS_NUM_BOXES = 1000
BBOX_XFORM_CLIP = 4.135166556742356

P = 56
T = 128
NB = 8
KC = NB * T
NMERGE = 704


def _decode_kernel(box_ref, cls_ref, anc_ref, img_ref, boxes_out, scores_out):
    deltas = box_ref[0]
    anchors = anc_ref[0]
    ya = (anchors[0] + anchors[2]) * 0.5
    xa = (anchors[1] + anchors[3]) * 0.5
    ha = anchors[2] - anchors[0]
    wa = anchors[3] - anchors[1]
    ty, tx = deltas[0], deltas[1]
    th = jnp.clip(deltas[2], -BBOX_XFORM_CLIP, BBOX_XFORM_CLIP)
    tw = jnp.clip(deltas[3], -BBOX_XFORM_CLIP, BBOX_XFORM_CLIP)
    h = jnp.exp(th) * ha
    w = jnp.exp(tw) * wa
    yc = ty * ha + ya
    xc = tx * wa + xa
    y1 = yc - h * 0.5
    x1 = xc - w * 0.5
    y2 = yc + h * 0.5
    x2 = xc + w * 0.5
    bi = pl.program_id(0)
    ih = img_ref[bi, 0]
    iw = img_ref[bi, 1]
    y1 = jnp.minimum(jnp.maximum(y1, 0.0), ih)
    x1 = jnp.minimum(jnp.maximum(x1, 0.0), iw)
    y2 = jnp.minimum(jnp.maximum(y2, 0.0), ih)
    x2 = jnp.minimum(jnp.maximum(x2, 0.0), iw)
    boxes_out[0] = jnp.stack([y1, x1, y2, x2], axis=0)
    scores_out[0] = jax.nn.sigmoid(cls_ref[0])


def _decode_pallas(box_t, cls_t, anc_t, image_shape):
    b, _, n = box_t.shape
    c = cls_t.shape[1]
    return pl.pallas_call(
        _decode_kernel,
        out_shape=(
            jax.ShapeDtypeStruct((b, 4, n), jnp.float32),
            jax.ShapeDtypeStruct((b, c, n), jnp.float32),
        ),
        grid=(b,),
        in_specs=[
            pl.BlockSpec((1, 4, n), lambda i: (i, 0, 0)),
            pl.BlockSpec((1, c, n), lambda i: (i, 0, 0)),
            pl.BlockSpec((1, 4, n), lambda i: (i, 0, 0)),
            pl.BlockSpec((b, 2), lambda i: (0, 0)),
        ],
        out_specs=(
            pl.BlockSpec((1, 4, n), lambda i: (i, 0, 0)),
            pl.BlockSpec((1, c, n), lambda i: (i, 0, 0)),
        ),
    )(box_t, cls_t, anc_t, image_shape)


def _bisect_kernel(s_ref, thr_ref, quota_ref):
    s = s_ref[...]
    bits = lax.bitcast_convert_type(s, jnp.int32)

    def it(_, lohi):
        lo, hi = lohi
        mid = lo + (hi - lo) // 2
        cnt = jnp.sum(jnp.where(bits >= mid, 1.0, 0.0), axis=1, keepdims=True)
        big = cnt >= float(PRE_NMS_NUM_BOXES)
        return jnp.where(big, mid, lo), jnp.where(big, hi, mid)

    lo0 = jnp.zeros((P, 1), jnp.int32)
    hi0 = jnp.full((P, 1), 0x3F800001, jnp.int32)
    lo, _ = lax.fori_loop(0, 31, it, (lo0, hi0))
    k_eff = jnp.maximum(lax.bitcast_convert_type(lo, jnp.float32),
                        SCORE_THRESHOLD)
    n_gt = jnp.sum(jnp.where(s > k_eff, 1.0, 0.0), axis=1, keepdims=True)
    quota = float(PRE_NMS_NUM_BOXES) - n_gt
    thr_ref[...] = jnp.broadcast_to(k_eff, (P, 16))
    quota_ref[...] = jnp.broadcast_to(quota, (P, 16))


def _bisect_pallas(scores56):
    return pl.pallas_call(
        _bisect_kernel,
        out_shape=(
            jax.ShapeDtypeStruct((P, 16), jnp.float32),
            jax.ShapeDtypeStruct((P, 16), jnp.float32),
        ),
    )(scores56)


def _sc_select_kernel(scores_hbm, boxflat_hbm, thr_hbm,
                      os_hbm, oy1_hbm, ox1_hbm, oy2_hbm, ox2_hbm,
                      sv, cand_s, cand_i, idx_tmp, bdst, sem):
    n = 20000
    nchunk = n // 16
    kc = KC
    info = plsc.get_sparse_core_info()
    nc = info.num_cores
    wid = lax.axis_index("s") * nc + lax.axis_index("c")
    lanes = lax.iota(jnp.int32, 16)
    out_refs = (os_hbm, oy1_hbm, ox1_hbm, oy2_hbm, ox2_hbm)

    for slot in range(2):
        p = wid if slot == 0 else jnp.where(wid < P - 32, wid + 32, wid)
        b = p // 7
        pltpu.sync_copy(scores_hbm.at[p], sv)
        pltpu.sync_copy(thr_hbm.at[p], bdst.at[pl.ds(0, 16)])
        thr_v = bdst[pl.ds(0, 16)]
        def initbuf(i, _):
            cand_s[pl.ds(i * 16, 16)] = jnp.full((16,), -1.0, jnp.float32)
            cand_i[pl.ds(i * 16, 16)] = jnp.zeros((16,), jnp.int32)
            return 0
        lax.fori_loop(0, (kc + 16) // 16, initbuf, 0)

        trash = lanes + kc

        def comp(i, og):
            x = sv[pl.ds(i * 16, 16)]
            iv = lanes + i * 16
            m = x >= thr_v
            g_i = m.astype(jnp.int32)
            pg = plsc.cumsum(g_i) - g_i
            dg = jnp.where(m, jnp.minimum(og + pg, kc - 1), trash)
            plsc.store_scatter(cand_s, [dg], x)
            plsc.store_scatter(cand_i, [dg], iv)
            return og + plsc.all_reduce_population_count(m)
        lax.fori_loop(0, nchunk, comp, jnp.zeros((16,), jnp.int32))

        pltpu.sync_copy(cand_s.at[pl.ds(0, kc)], os_hbm.at[p])
        for d in range(4):
            base = (b * 4 + d) * n

            def mkidx(v, _):
                idx_tmp[pl.ds(v * 16, 16)] = cand_i[pl.ds(v * 16, 16)] + base
                return 0
            lax.fori_loop(0, kc // 16, mkidx, 0)
            pltpu.async_copy(boxflat_hbm.at[idx_tmp], bdst, sem).wait()
            pltpu.sync_copy(bdst, out_refs[d + 1].at[p])


def _sc_gather_kernel(topi_hbm, boxflat_hbm,
                      oy1_hbm, ox1_hbm, oy2_hbm, ox2_hbm,
                      idxv, idx2, bdst, sem):
    n = 20000
    kc = KC
    info = plsc.get_sparse_core_info()
    nc = info.num_cores
    wid = lax.axis_index("s") * nc + lax.axis_index("c")
    out_refs = (oy1_hbm, ox1_hbm, oy2_hbm, ox2_hbm)
    for slot in range(2):
        p = wid if slot == 0 else jnp.where(wid < P - 32, wid + 32, wid)
        b = p // 7
        pltpu.sync_copy(topi_hbm.at[p], idxv)
        for d in range(4):
            base = (b * 4 + d) * n

            def mkidx(v, _):
                idx2[pl.ds(v * 16, 16)] = idxv[pl.ds(v * 16, 16)] + base
                return 0
            lax.fori_loop(0, kc // 16, mkidx, 0)
            pltpu.async_copy(boxflat_hbm.at[idx2], bdst, sem).wait()
            pltpu.sync_copy(bdst, out_refs[d].at[p])


def _sc_gather(topi, boxflat):
    mesh = plsc.VectorSubcoreMesh(core_axis_name="c", subcore_axis_name="s")
    out = jax.ShapeDtypeStruct((P, KC), jnp.float32)
    fn = functools.partial(
        pl.kernel,
        out_type=(out,) * 4,
        mesh=mesh,
        scratch_types=[
            pltpu.VMEM((KC,), jnp.int32),
            pltpu.VMEM((KC,), jnp.int32),
            pltpu.VMEM((KC,), jnp.float32),
            pltpu.SemaphoreType.DMA,
        ],
    )(_sc_gather_kernel)
    return fn(topi, boxflat)


def _sc_select(scores56, boxflat, thr16):
    mesh = plsc.VectorSubcoreMesh(core_axis_name="c", subcore_axis_name="s")
    out = jax.ShapeDtypeStruct((P, KC), jnp.float32)
    fn = functools.partial(
        pl.kernel,
        out_type=(out,) * 5,
        mesh=mesh,
        scratch_types=[
            pltpu.VMEM((20000,), jnp.float32),
            pltpu.VMEM((KC + 16,), jnp.float32),
            pltpu.VMEM((KC + 16,), jnp.int32),
            pltpu.VMEM((KC,), jnp.int32),
            pltpu.VMEM((KC,), jnp.float32),
            pltpu.SemaphoreType.DMA,
        ],
    )(_sc_select_kernel)
    return fn(scores56, boxflat, thr16)


def _sortperm_kernel(srow_ref, scol_ref, y1_ref, x1_ref, y2_ref, x2_ref,
                     out_ref):
    srow = srow_ref[0]
    scol = scol_ref[0]
    gt = jnp.where(srow > scol, 1.0, 0.0)
    iota_i = lax.broadcasted_iota(jnp.int32, (KC, KC), 1)
    iota_k = lax.broadcasted_iota(jnp.int32, (KC, KC), 0)
    tie = jnp.where((srow == scol) & (iota_i < iota_k), 1.0, 0.0)
    rank = jnp.sum(gt + tie, axis=1, keepdims=True)
    iota_o = lax.broadcasted_iota(jnp.int32, (1, KC), 1).astype(jnp.float32)
    oh = jnp.where(rank == iota_o, 1.0, 0.0)
    vals = jnp.concatenate(
        [srow, y1_ref[0], x1_ref[0], y2_ref[0], x2_ref[0]], axis=0)
    out_ref[0] = jax.lax.dot(vals, oh, precision=jax.lax.Precision.HIGHEST)


def _sortperm_pallas(s, y1, x1, y2, x2):
    row = pl.BlockSpec((1, 1, KC), lambda i: (i, 0, 0))
    col = pl.BlockSpec((1, KC, 1), lambda i: (i, 0, 0))
    return pl.pallas_call(
        _sortperm_kernel,
        out_shape=jax.ShapeDtypeStruct((P, 5, KC), jnp.float32),
        grid=(P,),
        in_specs=[row, col, row, row, row, row],
        out_specs=pl.BlockSpec((1, 5, KC), lambda i: (i, 0, 0)),
    )(s[:, None, :], s[:, :, None], y1[:, None, :], x1[:, None, :],
      y2[:, None, :], x2[:, None, :])


def _iou(ay1, ax1, ay2, ax2, by1, bx1, by2, bx2):
    yy1 = jnp.maximum(ay1, by1)
    xx1 = jnp.maximum(ax1, bx1)
    yy2 = jnp.minimum(ay2, by2)
    xx2 = jnp.minimum(ax2, bx2)
    inter = jnp.maximum(yy2 - yy1, 0.0) * jnp.maximum(xx2 - xx1, 0.0)
    a1 = (ay2 - ay1) * (ax2 - ax1)
    a2 = (by2 - by1) * (bx2 - bx1)
    return inter / (a1 + a2 - inter + 1e-8)


def _nms_kernel(s_ref, y1_ref, x1_ref, y2_ref, x2_ref,
                y1c_ref, x1c_ref, y2c_ref, x2c_ref,
                ss_ref, sy1_ref, sx1_ref, sy2_ref, sx2_ref):
    iota_k = lax.broadcasted_iota(jnp.int32, (T, T, 1), 0)
    iota_i = lax.broadcasted_iota(jnp.int32, (T, T, 1), 1)
    tri_ki = iota_i < iota_k
    iota_o = lax.broadcasted_iota(jnp.int32, (T, 1, 1), 0).astype(jnp.float32)

    def blk_cond(st):
        j, cnt, _, _, _, _, _ = st
        return (j < NB) & jnp.any(cnt < float(MAX_TOTAL_SIZE))

    def blk_body(st):
        j, cnt, ss, sy1, sx1, sy2, sx2 = st
        sb = s_ref[j]
        by1, bx1, by2, bx2 = y1_ref[j], x1_ref[j], y2_ref[j], x2_ref[j]
        cy1, cx1, cy2, cx2 = y1c_ref[j], x1c_ref[j], y2c_ref[j], x2c_ref[j]
        active0 = sb >= 0.0
        iou_ps = _iou(cy1, cx1, cy2, cx2,
                      sy1[None], sx1[None], sy2[None], sx2[None])
        active0 = active0 & ~jnp.any(iou_ps > NMS_IOU_THRESHOLD, axis=1)
        iou_bb = _iou(cy1, cx1, cy2, cx2,
                      by1[None], bx1[None], by2[None], bx2[None])
        can_supf = jnp.where((iou_bb > NMS_IOU_THRESHOLD) & tri_ki, 1.0, 0.0)

        def f_cond(fs):
            _, changed, it = fs
            return (changed > 0) & (it < T)

        def f_body(fs):
            Sf, _, it = fs
            supped = jnp.max(Sf[None] * can_supf, axis=1)
            Snf = jnp.where(active0 & (supped <= 0.0), 1.0, 0.0)
            chg = jnp.any(Snf != Sf).astype(jnp.int32)
            return Snf, chg, it + 1

        Sf, _, _ = lax.while_loop(
            f_cond, f_body,
            (jnp.where(active0, 1.0, 0.0), jnp.int32(1), jnp.int32(0)))
        S = Sf > 0.0
        pos_in = jnp.sum(jnp.where(tri_ki, Sf[None], 0.0), axis=1)
        pos = cnt + pos_in
        emit = S & (pos < float(MAX_TOTAL_SIZE))
        ohf = jnp.where(emit[None] & (pos[None] == iota_o), 1.0, 0.0)
        hit = jnp.sum(ohf, axis=1) > 0.0

        def emit_val(v):
            return jnp.sum(ohf * v[None], axis=1)

        ss = jnp.where(hit, emit_val(sb), ss)
        sy1 = jnp.where(hit, emit_val(by1), sy1)
        sx1 = jnp.where(hit, emit_val(bx1), sx1)
        sy2 = jnp.where(hit, emit_val(by2), sy2)
        sx2 = jnp.where(hit, emit_val(bx2), sx2)
        cnt = cnt + jnp.sum(Sf, axis=0, keepdims=True)
        return j + 1, cnt, ss, sy1, sx1, sy2, sx2

    init = (jnp.int32(0), jnp.zeros((1, P), jnp.float32),
            jnp.full((T, P), -1.0, jnp.float32),
            jnp.zeros((T, P), jnp.float32), jnp.zeros((T, P), jnp.float32),
            jnp.zeros((T, P), jnp.float32), jnp.zeros((T, P), jnp.float32))
    _, _, ss, sy1, sx1, sy2, sx2 = lax.while_loop(blk_cond, blk_body, init)
    ss_ref[...] = ss
    sy1_ref[...] = sy1
    sx1_ref[...] = sx1
    sy2_ref[...] = sy2
    sx2_ref[...] = sx2


def _nms_pallas(s3, planes3):
    planes4 = [q[:, :, None, :] for q in planes3]
    out = jax.ShapeDtypeStruct((T, P), jnp.float32)
    return pl.pallas_call(
        _nms_kernel,
        out_shape=(out,) * 5,
    )(s3, *planes3, *planes4)


def _merge_kernel(srow_ref, scol_ref, y1_ref, x1_ref, y2_ref, x2_ref,
                  cls_ref, vals_ref, valid_ref):
    srow = srow_ref[0]
    scol = scol_ref[0]
    gt = jnp.where(srow > scol, 1.0, 0.0)
    iota_j = lax.broadcasted_iota(jnp.int32, (NMERGE, NMERGE), 1)
    iota_i = lax.broadcasted_iota(jnp.int32, (NMERGE, NMERGE), 0)
    tie = jnp.where((srow == scol) & (iota_j < iota_i), 1.0, 0.0)
    rank = jnp.sum(gt + tie, axis=1, keepdims=True)
    iota_o = lax.broadcasted_iota(jnp.int32, (1, T), 1).astype(jnp.float32)
    oh = jnp.where(rank == iota_o, 1.0, 0.0)

    def emit(vcol_ref):
        return jnp.sum(oh * vcol_ref[0], axis=0, keepdims=True)

    out_s = emit(scol_ref)
    vals_ref[0, 0:1, :] = out_s
    vals_ref[0, 1:2, :] = emit(y1_ref)
    vals_ref[0, 2:3, :] = emit(x1_ref)
    vals_ref[0, 3:4, :] = emit(y2_ref)
    vals_ref[0, 4:5, :] = emit(x2_ref)
    vals_ref[0, 5:6, :] = emit(cls_ref)
    nvalid = jnp.sum(jnp.where((out_s > -1.0) & (iota_o < float(MAX_TOTAL_SIZE)),
                               1.0, 0.0), axis=1, keepdims=True)
    valid_ref[0] = jnp.broadcast_to(nvalid, (1, T)).astype(jnp.int32)


def _merge_pallas(srow, scol, y1c, x1c, y2c, x2c, clsc):
    b = srow.shape[0]
    row_spec = pl.BlockSpec((1, 1, NMERGE), lambda i: (i, 0, 0))
    col_spec = pl.BlockSpec((1, NMERGE, 1), lambda i: (i, 0, 0))
    return pl.pallas_call(
        _merge_kernel,
        out_shape=(
            jax.ShapeDtypeStruct((b, 6, T), jnp.float32),
            jax.ShapeDtypeStruct((b, 1, T), jnp.int32),
        ),
        grid=(b,),
        in_specs=[row_spec] + [col_spec] * 6,
        out_specs=(
            pl.BlockSpec((1, 6, T), lambda i: (i, 0, 0)),
            pl.BlockSpec((1, 1, T), lambda i: (i, 0, 0)),
        ),
    )(srow, scol, y1c, x1c, y2c, x2c, clsc)


@jax.jit
def kernel(box_outputs, class_outputs, anchor_boxes, image_shape):
    b, n, _ = box_outputs.shape
    boxes_t, scores_t = _decode_pallas(
        box_outputs.transpose(0, 2, 1), class_outputs.transpose(0, 2, 1),
        anchor_boxes.transpose(0, 2, 1), image_shape)
    scores56 = scores_t[:, 1:, :].reshape(P, n)
    k = min(n, PRE_NMS_NUM_BOXES)
    topv, topi = lax.top_k(scores56, k)
    pad = KC - k
    sv = jnp.where(topv >= SCORE_THRESHOLD, topv, -1.0)
    sv = jnp.concatenate([sv, jnp.full((P, pad), -1.0, jnp.float32)], axis=1)
    topi = jnp.concatenate([topi, jnp.zeros((P, pad), jnp.int32)], axis=1)
    boxflat = boxes_t.reshape(b * 4 * n)
    cy1, cx1, cy2, cx2 = _sc_gather(topi, boxflat)

    def to3(a):
        return a.reshape(P, NB, T).transpose(1, 2, 0)

    ss, sy1, sx1, sy2, sx2 = _nms_pallas(
        to3(sv), [to3(q) for q in (cy1, cx1, cy2, cx2)])
    ss, sy1, sx1, sy2, sx2 = (a.T for a in (ss, sy1, sx1, sy2, sx2))

    def to_m(a):
        return a.reshape(b, 7, T)[:, :, :MAX_TOTAL_SIZE] \
                .reshape(b, 7 * MAX_TOTAL_SIZE)

    s_m = to_m(ss)
    cls_m = jnp.broadcast_to(
        jnp.arange(7, dtype=jnp.float32)[None, :, None],
        (b, 7, MAX_TOTAL_SIZE)).reshape(b, 7 * MAX_TOTAL_SIZE)
    padm = NMERGE - 7 * MAX_TOTAL_SIZE
    s_m = jnp.concatenate([s_m, jnp.full((b, padm), -2.0, jnp.float32)], axis=1)

    def padz(a):
        return jnp.concatenate([a, jnp.zeros((b, padm), jnp.float32)], axis=1)

    cols = [s_m[:, :, None]] + \
           [padz(to_m(q))[:, :, None] for q in (sy1, sx1, sy2, sx2)] + \
           [padz(cls_m)[:, :, None]]
    vals, valid = _merge_pallas(s_m[:, None, :], *cols)
    final_scores = vals[:, 0, :MAX_TOTAL_SIZE]
    final_boxes = vals[:, 1:5, :MAX_TOTAL_SIZE].transpose(0, 2, 1)
    final_classes = vals[:, 5, :MAX_TOTAL_SIZE].astype(jnp.int32)
    valid_detections = valid[:, 0, 0]
    return final_boxes, final_scores, final_classes, valid_detections


# speedup vs baseline: 4.0061x; 1.0004x over previous
import functools
import jax
import jax.numpy as jnp
from jax import lax
from jax.experimental import pallas as pl
from jax.experimental.pallas import tpu as pltpu
from jax.experimental.pallas import tpu_sc as plsc

NUM_CLASSES = 8
MAX_TOTAL_SIZE = 100
NMS_IOU_THRESHOLD = 0.5
SCORE_THRESHOLD = 0.05
PRE_NMS_NUM_BOXES = 1000
BBOX_XFORM_CLIP = 4.135166556742356

P = 56
T = 128
NB = 8
KC = NB * T
NMERGE = 704


def _decode_kernel(box_ref, cls_ref, anc_ref, img_ref, boxes_out, scores_out):
    deltas = box_ref[0]
    anchors = anc_ref[0]
    ya = (anchors[0] + anchors[2]) * 0.5
    xa = (anchors[1] + anchors[3]) * 0.5
    ha = anchors[2] - anchors[0]
    wa = anchors[3] - anchors[1]
    ty, tx = deltas[0], deltas[1]
    th = jnp.clip(deltas[2], -BBOX_XFORM_CLIP, BBOX_XFORM_CLIP)
    tw = jnp.clip(deltas[3], -BBOX_XFORM_CLIP, BBOX_XFORM_CLIP)
    h = jnp.exp(th) * ha
    w = jnp.exp(tw) * wa
    yc = ty * ha + ya
    xc = tx * wa + xa
    y1 = yc - h * 0.5
    x1 = xc - w * 0.5
    y2 = yc + h * 0.5
    x2 = xc + w * 0.5
    bi = pl.program_id(0)
    ih = img_ref[bi, 0]
    iw = img_ref[bi, 1]
    y1 = jnp.minimum(jnp.maximum(y1, 0.0), ih)
    x1 = jnp.minimum(jnp.maximum(x1, 0.0), iw)
    y2 = jnp.minimum(jnp.maximum(y2, 0.0), ih)
    x2 = jnp.minimum(jnp.maximum(x2, 0.0), iw)
    boxes_out[0] = jnp.stack([y1, x1, y2, x2], axis=0)
    scores_out[0] = jax.nn.sigmoid(cls_ref[0])


def _decode_pallas(box_t, cls_t, anc_t, image_shape):
    b, _, n = box_t.shape
    c = cls_t.shape[1]
    return pl.pallas_call(
        _decode_kernel,
        out_shape=(
            jax.ShapeDtypeStruct((b, 4, n), jnp.float32),
            jax.ShapeDtypeStruct((b, c, n), jnp.float32),
        ),
        grid=(b,),
        in_specs=[
            pl.BlockSpec((1, 4, n), lambda i: (i, 0, 0)),
            pl.BlockSpec((1, c, n), lambda i: (i, 0, 0)),
            pl.BlockSpec((1, 4, n), lambda i: (i, 0, 0)),
            pl.BlockSpec((b, 2), lambda i: (0, 0)),
        ],
        out_specs=(
            pl.BlockSpec((1, 4, n), lambda i: (i, 0, 0)),
            pl.BlockSpec((1, c, n), lambda i: (i, 0, 0)),
        ),
    )(box_t, cls_t, anc_t, image_shape)


def _sc_gather_kernel(topi_hbm, boxflat_hbm,
                      oy1_hbm, ox1_hbm, oy2_hbm, ox2_hbm,
                      idxv, idx2, bdst, sem):
    n = 20000
    kc = KC
    info = plsc.get_sparse_core_info()
    nc = info.num_cores
    wid = lax.axis_index("s") * nc + lax.axis_index("c")
    out_refs = (oy1_hbm, ox1_hbm, oy2_hbm, ox2_hbm)
    for slot in range(2):
        p = wid if slot == 0 else jnp.where(wid < P - 32, wid + 32, wid)
        b = p // 7
        pltpu.sync_copy(topi_hbm.at[p], idxv)
        for d in range(4):
            base = (b * 4 + d) * n

            def mkidx(v, _):
                idx2[pl.ds(v * 16, 16)] = idxv[pl.ds(v * 16, 16)] + base
                return 0
            lax.fori_loop(0, kc // 16, mkidx, 0)
            pltpu.async_copy(boxflat_hbm.at[idx2], bdst, sem).wait()
            pltpu.sync_copy(bdst, out_refs[d].at[p])


def _sc_gather(topi, boxflat):
    mesh = plsc.VectorSubcoreMesh(core_axis_name="c", subcore_axis_name="s")
    out = jax.ShapeDtypeStruct((P, KC), jnp.float32)
    fn = functools.partial(
        pl.kernel,
        out_type=(out,) * 4,
        mesh=mesh,
        scratch_types=[
            pltpu.VMEM((KC,), jnp.int32),
            pltpu.VMEM((KC,), jnp.int32),
            pltpu.VMEM((KC,), jnp.float32),
            pltpu.SemaphoreType.DMA,
        ],
    )(_sc_gather_kernel)
    return fn(topi, boxflat)


def _iou(ay1, ax1, ay2, ax2, by1, bx1, by2, bx2):
    yy1 = jnp.maximum(ay1, by1)
    xx1 = jnp.maximum(ax1, bx1)
    yy2 = jnp.minimum(ay2, by2)
    xx2 = jnp.minimum(ax2, bx2)
    inter = jnp.maximum(yy2 - yy1, 0.0) * jnp.maximum(xx2 - xx1, 0.0)
    a1 = (ay2 - ay1) * (ax2 - ax1)
    a2 = (by2 - by1) * (bx2 - bx1)
    return inter / (a1 + a2 - inter + 1e-8)


def _nms_kernel(s_ref, y1_ref, x1_ref, y2_ref, x2_ref,
                y1c_ref, x1c_ref, y2c_ref, x2c_ref,
                ss_ref, sy1_ref, sx1_ref, sy2_ref, sx2_ref):
    iota_k = lax.broadcasted_iota(jnp.int32, (T, T, 1), 0)
    iota_i = lax.broadcasted_iota(jnp.int32, (T, T, 1), 1)
    tri_ki = iota_i < iota_k
    iota_o = lax.broadcasted_iota(jnp.int32, (T, 1, 1), 0).astype(jnp.float32)

    def blk_cond(st):
        j, cnt, _, _, _, _, _ = st
        return (j < NB) & jnp.any(cnt < float(MAX_TOTAL_SIZE))

    def blk_body(st):
        j, cnt, ss, sy1, sx1, sy2, sx2 = st
        sb = s_ref[j]
        by1, bx1, by2, bx2 = y1_ref[j], x1_ref[j], y2_ref[j], x2_ref[j]
        cy1, cx1, cy2, cx2 = y1c_ref[j], x1c_ref[j], y2c_ref[j], x2c_ref[j]
        active0 = sb >= 0.0
        iou_ps = _iou(cy1, cx1, cy2, cx2,
                      sy1[None], sx1[None], sy2[None], sx2[None])
        active0 = active0 & ~jnp.any(iou_ps > NMS_IOU_THRESHOLD, axis=1)
        iou_bb = _iou(cy1, cx1, cy2, cx2,
                      by1[None], bx1[None], by2[None], bx2[None])
        can_supf = jnp.where((iou_bb > NMS_IOU_THRESHOLD) & tri_ki, 1.0, 0.0)

        def f_cond(fs):
            _, changed, it = fs
            return (changed > 0) & (it < T)

        def f_body(fs):
            Sf, _, it = fs
            supped = jnp.max(Sf[None] * can_supf, axis=1)
            Snf = jnp.where(active0 & (supped <= 0.0), 1.0, 0.0)
            chg = jnp.any(Snf != Sf).astype(jnp.int32)
            return Snf, chg, it + 1

        Sf, _, _ = lax.while_loop(
            f_cond, f_body,
            (jnp.where(active0, 1.0, 0.0), jnp.int32(1), jnp.int32(0)))
        S = Sf > 0.0
        pos_in = jnp.sum(jnp.where(tri_ki, Sf[None], 0.0), axis=1)
        pos = cnt + pos_in
        emit = S & (pos < float(MAX_TOTAL_SIZE))
        ohf = jnp.where(emit[None] & (pos[None] == iota_o), 1.0, 0.0)
        hit = jnp.sum(ohf, axis=1) > 0.0

        def emit_val(v):
            return jnp.sum(ohf * v[None], axis=1)

        ss = jnp.where(hit, emit_val(sb), ss)
        sy1 = jnp.where(hit, emit_val(by1), sy1)
        sx1 = jnp.where(hit, emit_val(bx1), sx1)
        sy2 = jnp.where(hit, emit_val(by2), sy2)
        sx2 = jnp.where(hit, emit_val(bx2), sx2)
        cnt = cnt + jnp.sum(Sf, axis=0, keepdims=True)
        return j + 1, cnt, ss, sy1, sx1, sy2, sx2

    init = (jnp.int32(0), jnp.zeros((1, P), jnp.float32),
            jnp.full((T, P), -1.0, jnp.float32),
            jnp.zeros((T, P), jnp.float32), jnp.zeros((T, P), jnp.float32),
            jnp.zeros((T, P), jnp.float32), jnp.zeros((T, P), jnp.float32))
    _, _, ss, sy1, sx1, sy2, sx2 = lax.while_loop(blk_cond, blk_body, init)
    ss_ref[...] = ss
    sy1_ref[...] = sy1
    sx1_ref[...] = sx1
    sy2_ref[...] = sy2
    sx2_ref[...] = sx2


def _nms_pallas(s3, planes3):
    planes4 = [q[:, :, None, :] for q in planes3]
    out = jax.ShapeDtypeStruct((T, P), jnp.float32)
    return pl.pallas_call(
        _nms_kernel,
        out_shape=(out,) * 5,
    )(s3, *planes3, *planes4)


def _merge_kernel(srow_ref, scol_ref, y1_ref, x1_ref, y2_ref, x2_ref,
                  cls_ref, vals_ref, valid_ref):
    srow = srow_ref[0]
    scol = scol_ref[0]
    gt = jnp.where(srow > scol, 1.0, 0.0)
    iota_j = lax.broadcasted_iota(jnp.int32, (NMERGE, NMERGE), 1)
    iota_i = lax.broadcasted_iota(jnp.int32, (NMERGE, NMERGE), 0)
    tie = jnp.where((srow == scol) & (iota_j < iota_i), 1.0, 0.0)
    rank = jnp.sum(gt + tie, axis=1, keepdims=True)
    iota_o = lax.broadcasted_iota(jnp.int32, (1, T), 1).astype(jnp.float32)
    oh = jnp.where(rank == iota_o, 1.0, 0.0)

    def emit(vcol_ref):
        return jnp.sum(oh * vcol_ref[0], axis=0, keepdims=True)

    out_s = emit(scol_ref)
    vals_ref[0, 0:1, :] = out_s
    vals_ref[0, 1:2, :] = emit(y1_ref)
    vals_ref[0, 2:3, :] = emit(x1_ref)
    vals_ref[0, 3:4, :] = emit(y2_ref)
    vals_ref[0, 4:5, :] = emit(x2_ref)
    vals_ref[0, 5:6, :] = emit(cls_ref)
    nvalid = jnp.sum(jnp.where((out_s > -1.0) & (iota_o < float(MAX_TOTAL_SIZE)),
                               1.0, 0.0), axis=1, keepdims=True)
    valid_ref[0] = jnp.broadcast_to(nvalid, (1, T)).astype(jnp.int32)


def _merge_pallas(srow, scol, y1c, x1c, y2c, x2c, clsc):
    b = srow.shape[0]
    row_spec = pl.BlockSpec((1, 1, NMERGE), lambda i: (i, 0, 0))
    col_spec = pl.BlockSpec((1, NMERGE, 1), lambda i: (i, 0, 0))
    return pl.pallas_call(
        _merge_kernel,
        out_shape=(
            jax.ShapeDtypeStruct((b, 6, T), jnp.float32),
            jax.ShapeDtypeStruct((b, 1, T), jnp.int32),
        ),
        grid=(b,),
        in_specs=[row_spec] + [col_spec] * 6,
        out_specs=(
            pl.BlockSpec((1, 6, T), lambda i: (i, 0, 0)),
            pl.BlockSpec((1, 1, T), lambda i: (i, 0, 0)),
        ),
    )(srow, scol, y1c, x1c, y2c, x2c, clsc)


@jax.jit
def kernel(box_outputs, class_outputs, anchor_boxes, image_shape):
    b, n, _ = box_outputs.shape
    boxes_t, scores_t = _decode_pallas(
        box_outputs.transpose(0, 2, 1), class_outputs.transpose(0, 2, 1),
        anchor_boxes.transpose(0, 2, 1), image_shape)
    scores56 = scores_t[:, 1:, :].reshape(P, n)
    k = min(n, PRE_NMS_NUM_BOXES)
    topv, topi = lax.top_k(scores56, k)
    pad = KC - k
    sv = jnp.where(topv >= SCORE_THRESHOLD, topv, -1.0)
    sv = jnp.concatenate([sv, jnp.full((P, pad), -1.0, jnp.float32)], axis=1)
    topi = jnp.concatenate([topi, jnp.zeros((P, pad), jnp.int32)], axis=1)
    boxflat = boxes_t.reshape(b * 4 * n)
    cy1, cx1, cy2, cx2 = _sc_gather(topi, boxflat)

    def to3(a):
        return a.reshape(P, NB, T).transpose(1, 2, 0)

    ss, sy1, sx1, sy2, sx2 = _nms_pallas(
        to3(sv), [to3(q) for q in (cy1, cx1, cy2, cx2)])
    ss, sy1, sx1, sy2, sx2 = (a.T for a in (ss, sy1, sx1, sy2, sx2))

    def to_m(a):
        return a.reshape(b, 7, T)[:, :, :MAX_TOTAL_SIZE] \
                .reshape(b, 7 * MAX_TOTAL_SIZE)

    s_m = to_m(ss)
    cls_m = jnp.broadcast_to(
        jnp.arange(7, dtype=jnp.float32)[None, :, None],
        (b, 7, MAX_TOTAL_SIZE)).reshape(b, 7 * MAX_TOTAL_SIZE)
    padm = NMERGE - 7 * MAX_TOTAL_SIZE
    s_m = jnp.concatenate([s_m, jnp.full((b, padm), -2.0, jnp.float32)], axis=1)

    def padz(a):
        return jnp.concatenate([a, jnp.zeros((b, padm), jnp.float32)], axis=1)

    cols = [s_m[:, :, None]] + \
           [padz(to_m(q))[:, :, None] for q in (sy1, sx1, sy2, sx2)] + \
           [padz(cls_m)[:, :, None]]
    vals, valid = _merge_pallas(s_m[:, None, :], *cols)
    final_scores = vals[:, 0, :MAX_TOTAL_SIZE]
    final_boxes = vals[:, 1:5, :MAX_TOTAL_SIZE].transpose(0, 2, 1)
    final_classes = vals[:, 5, :MAX_TOTAL_SIZE].astype(jnp.int32)
    valid_detections = valid[:, 0, 0]
    return final_boxes, final_scores, final_classes, valid_detections
